# baseline (speedup 1.0000x reference)
import jax
import jax.numpy as jnp
from jax import lax
from jax.experimental import pallas as pl
from jax.experimental.pallas import tpu as pltpu
from jax.experimental.pallas import tpu_sc as plsc

N = 20000
G = 128
B = 8
RPW = 640
LAST_BASE = N - RPW
UF = 8
SEG = G + 2 * UF
NSEG = B * SEG // 16 + 1
CHUNK = 128
NCH = RPW // CHUNK


def _log_f32(x):
    bits = lax.bitcast_convert_type(x, jnp.int32)
    e = (bits >> 23) - 127
    mb = (bits & 0x007FFFFF) | 0x3F800000
    m = lax.bitcast_convert_type(mb, jnp.float32)
    big = m > 1.41421356
    m = jnp.where(big, m * 0.5, m)
    ef = e.astype(jnp.float32) + jnp.where(big, 1.0, 0.0)
    s = (m - 1.0) / (m + 1.0)
    t = s * s
    p = 2.0 + t * (0.6666666666 + t * (0.4 + t * 0.2857142857))
    return ef * 0.6931471805599453 + s * p


def _body(rois_hbm, rb_hbm, gt_hbm, gb_hbm,
          lab_hbm, dl_hbm, bw_hbm,
          roiq0_v, roiq1_v, rb_v, gts_v, gb_v,
          gx1_v, gy1_v, gx2_v, gy2_v, glb_v,
          plist_v, gcnt_v,
          x1s_v, y1s_v, x2s_v, y2s_v, ags_v, lbs_v,
          lab_v, dlq0_v, dlq1_v, bwq0_v, bwq1_v, fill_s,
          rsem0, rsem1, dsem0, dsem1, wsem0, wsem1):
    wid = lax.axis_index("s") * 2 + lax.axis_index("c")
    base = jnp.minimum(wid * RPW, LAST_BASE)
    pltpu.sync_copy(rb_hbm.at[pl.ds(base, RPW)], rb_v)
    pltpu.sync_copy(gt_hbm, gts_v)
    pltpu.sync_copy(gb_hbm, gb_v)

    iota = lax.iota(jnp.int32, 16)
    zero = jnp.zeros((16,), jnp.float32)
    izero = jnp.zeros((16,), jnp.int32)
    ione = jnp.ones((16,), jnp.int32)
    c0 = izero
    c1 = ione
    c2 = jnp.full((16,), 2, jnp.int32)
    c3 = jnp.full((16,), 3, jnp.int32)
    c4 = jnp.full((16,), 4, jnp.int32)

    for c in range(G // 16):
        sl = pl.ds(c * 16, 16)
        rows = iota + c * 16
        x1 = plsc.load_gather(gts_v, [rows, c0])
        y1 = plsc.load_gather(gts_v, [rows, c1])
        x2 = plsc.load_gather(gts_v, [rows, c2])
        y2 = plsc.load_gather(gts_v, [rows, c3])
        gx1_v[sl] = x1
        gy1_v[sl] = y1
        gx2_v[sl] = x2
        gy2_v[sl] = y2
        glb_v[sl] = plsc.load_gather(gts_v, [rows, c4])
    zsl = pl.ds(G, 16)
    gx1_v[zsl] = zero
    gy1_v[zsl] = zero
    gx2_v[zsl] = zero
    gy2_v[zsl] = zero
    glb_v[zsl] = zero

    counts = izero
    for c in range(G // 16):
        gbc = gb_v[pl.ds(c * 16, 16)]
        for b in range(B):
            m = gbc == b
            cnt = plsc.all_reduce_population_count(m)
            counts = counts + jnp.where(iota == b, cnt, izero)
    gcnt_v[...] = counts
    mmax = jnp.max(counts)
    stride = ((mmax + (UF - 1)) // UF) * UF

    def initp(i, carry):
        plist_v[pl.ds(i * 16, 16)] = jnp.full((16,), G, jnp.int32)
        return carry
    lax.fori_loop(0, NSEG, initp, 0)
    for b in range(B):
        fill_s[b] = b * stride
    for c in range(G // 16):
        gbc = gb_v[pl.ds(c * 16, 16)]
        lanes = iota + c * 16
        for b in range(B):
            m = gbc == b
            ones = jnp.where(m, ione, izero)
            pc = plsc.cumsum(ones)
            fb = fill_s[b]
            plsc.store_scatter(plist_v, [pc - 1 + fb], lanes, mask=m)
            fill_s[b] = fb + jnp.max(pc)

    def permute(i, carry):
        sl = pl.ds(i * 16, 16)
        gi = plist_v[sl]
        x1 = plsc.load_gather(gx1_v, [gi])
        y1 = plsc.load_gather(gy1_v, [gi])
        x2 = plsc.load_gather(gx2_v, [gi])
        y2 = plsc.load_gather(gy2_v, [gi])
        x1s_v[sl] = x1
        y1s_v[sl] = y1
        x2s_v[sl] = x2
        y2s_v[sl] = y2
        lbs_v[sl] = plsc.load_gather(glb_v, [gi])
        ags_v[sl] = (x2 - x1) * (y2 - y1)
        return carry
    lax.fori_loop(0, NSEG, permute, 0)

    stride_v = jnp.broadcast_to(stride, (16,))

    rbufs = (roiq0_v, roiq1_v)
    dbufs = (dlq0_v, dlq1_v)
    wbufs = (bwq0_v, bwq1_v)
    rsems = (rsem0, rsem1)
    dsems = (dsem0, dsem1)
    wsems = (wsem0, wsem1)
    rdesc = [None, None]
    ddesc = [None, None]
    wdesc = [None, None]

    rdesc[0] = pltpu.async_copy(
        rois_hbm.at[pl.ds(base, CHUNK)], rbufs[0], rsems[0])
    for q in range(NCH):
        A = q & 1
        if q + 1 < NCH:
            rdesc[1 - A] = pltpu.async_copy(
                rois_hbm.at[pl.ds(base + (q + 1) * CHUNK, CHUNK)],
                rbufs[1 - A], rsems[1 - A])
        rdesc[A].wait()
        if q >= 2:
            ddesc[A].wait()
            wdesc[A].wait()
        roiq_v = rbufs[A]
        dlq_v = dbufs[A]
        bwq_v = wbufs[A]

        def group(s, carry, q=q, roiq_v=roiq_v, dlq_v=dlq_v, bwq_v=bwq_v):
            rbase = q * CHUNK + s * 16
            rloc = iota + s * 16
            rx1 = plsc.load_gather(roiq_v, [rloc, c0])
            ry1 = plsc.load_gather(roiq_v, [rloc, c1])
            rx2 = plsc.load_gather(roiq_v, [rloc, c2])
            ry2 = plsc.load_gather(roiq_v, [rloc, c3])
            rb = rb_v[pl.ds(rbase, 16)]
            ar = (rx2 - rx1) * (ry2 - ry1)
            seg = rb * stride_v
            gseg_cnt = plsc.load_gather(gcnt_v, [rb])
            tmax = jnp.max(gseg_cnt)

            def gstep(i, st):
                ib, db, ab = st
                t0 = i * UF
                for k in range(UF):
                    pos = seg + jnp.broadcast_to(t0 + k, (16,))
                    gx1 = plsc.load_gather(x1s_v, [pos])
                    gy1 = plsc.load_gather(y1s_v, [pos])
                    gx2 = plsc.load_gather(x2s_v, [pos])
                    gy2 = plsc.load_gather(y2s_v, [pos])
                    ag = plsc.load_gather(ags_v, [pos])
                    iw = jnp.maximum(
                        jnp.minimum(rx2, gx2) - jnp.maximum(rx1, gx1), 0.0)
                    ih = jnp.maximum(
                        jnp.minimum(ry2, gy2) - jnp.maximum(ry1, gy1), 0.0)
                    inter = iw * ih
                    den = (ar + ag) - inter
                    upd = inter * db > ib * den
                    ib = jnp.where(upd, inter, ib)
                    db = jnp.where(upd, den, db)
                    ab = jnp.where(upd, pos, ab)
                return (ib, db, ab)

            nsteps = (tmax + (UF - 1)) // UF
            ib, db, ab = lax.fori_loop(
                0, nsteps, gstep,
                (zero, jnp.ones((16,), jnp.float32), izero))

            fg = (ib + ib) >= db
            mx1 = plsc.load_gather(x1s_v, [ab])
            my1 = plsc.load_gather(y1s_v, [ab])
            mx2 = plsc.load_gather(x2s_v, [ab])
            my2 = plsc.load_gather(y2s_v, [ab])
            mlb = plsc.load_gather(lbs_v, [ab])
            pw = rx2 - rx1
            ph = ry2 - ry1
            pcx = rx1 + 0.5 * pw
            pcy = ry1 + 0.5 * ph
            gw = mx2 - mx1
            gh = my2 - my1
            gcx = mx1 + 0.5 * gw
            gcy = my1 + 0.5 * gh
            pwe = pw + 1e-12
            phe = ph + 1e-12
            dx = (gcx - pcx) / pwe
            dy = (gcy - pcy) / phe
            dw = _log_f32(gw / pwe + 1e-12)
            dh = _log_f32(gh / phe + 1e-12)
            lab_v[pl.ds(rbase, 16)] = jnp.where(fg, mlb, zero)
            plsc.store_scatter(dlq_v, [rloc, c0], jnp.where(fg, dx, zero))
            plsc.store_scatter(dlq_v, [rloc, c1], jnp.where(fg, dy, zero))
            plsc.store_scatter(dlq_v, [rloc, c2], jnp.where(fg, dw, zero))
            plsc.store_scatter(dlq_v, [rloc, c3], jnp.where(fg, dh, zero))
            one = jnp.where(fg, jnp.ones((16,), jnp.float32), zero)
            plsc.store_scatter(bwq_v, [rloc, c0], one)
            plsc.store_scatter(bwq_v, [rloc, c1], one)
            plsc.store_scatter(bwq_v, [rloc, c2], one)
            plsc.store_scatter(bwq_v, [rloc, c3], one)
            return carry

        lax.fori_loop(0, CHUNK // 16, group, 0)
        crow = base + q * CHUNK
        ddesc[A] = pltpu.async_copy(dlq_v, dl_hbm.at[pl.ds(crow, CHUNK)], dsems[A])
        wdesc[A] = pltpu.async_copy(bwq_v, bw_hbm.at[pl.ds(crow, CHUNK)], wsems[A])

    ddesc[(NCH - 1) & 1].wait()
    wdesc[(NCH - 1) & 1].wait()
    ddesc[(NCH - 2) & 1].wait()
    wdesc[(NCH - 2) & 1].wait()

    pltpu.sync_copy(lab_v, lab_hbm.at[pl.ds(base, RPW)])


def kernel(rois, roi_batch_inds, gt_boxes, gt_batch_inds):
    mesh = plsc.VectorSubcoreMesh(core_axis_name="c", subcore_axis_name="s")
    run = pl.kernel(
        _body,
        out_type=(jax.ShapeDtypeStruct((N,), jnp.float32),
                  jax.ShapeDtypeStruct((N, 4), jnp.float32),
                  jax.ShapeDtypeStruct((N, 4), jnp.float32)),
        mesh=mesh,
        compiler_params=pltpu.CompilerParams(needs_layout_passes=False),
        scratch_types=[
            pltpu.VMEM((CHUNK, 5), jnp.float32),
            pltpu.VMEM((CHUNK, 5), jnp.float32),
            pltpu.VMEM((RPW,), jnp.int32),
            pltpu.VMEM((G, 5), jnp.float32),
            pltpu.VMEM((G,), jnp.int32),
            pltpu.VMEM((G + 16,), jnp.float32),
            pltpu.VMEM((G + 16,), jnp.float32),
            pltpu.VMEM((G + 16,), jnp.float32),
            pltpu.VMEM((G + 16,), jnp.float32),
            pltpu.VMEM((G + 16,), jnp.float32),
            pltpu.VMEM((NSEG * 16,), jnp.int32),
            pltpu.VMEM((16,), jnp.int32),
            pltpu.VMEM((NSEG * 16,), jnp.float32),
            pltpu.VMEM((NSEG * 16,), jnp.float32),
            pltpu.VMEM((NSEG * 16,), jnp.float32),
            pltpu.VMEM((NSEG * 16,), jnp.float32),
            pltpu.VMEM((NSEG * 16,), jnp.float32),
            pltpu.VMEM((NSEG * 16,), jnp.float32),
            pltpu.VMEM((RPW,), jnp.float32),
            pltpu.VMEM((CHUNK, 4), jnp.float32),
            pltpu.VMEM((CHUNK, 4), jnp.float32),
            pltpu.VMEM((CHUNK, 4), jnp.float32),
            pltpu.VMEM((CHUNK, 4), jnp.float32),
            pltpu.SMEM((B,), jnp.int32),
            pltpu.SemaphoreType.DMA,
            pltpu.SemaphoreType.DMA,
            pltpu.SemaphoreType.DMA,
            pltpu.SemaphoreType.DMA,
            pltpu.SemaphoreType.DMA,
            pltpu.SemaphoreType.DMA,
        ],
    )
    lab, dl, bw = run(rois, roi_batch_inds, gt_boxes, gt_batch_inds)
    return lab, dl, bw


# speedup vs baseline: 1.1312x; 1.1312x over previous
import jax
import jax.numpy as jnp
from jax import lax
from jax.experimental import pallas as pl
from jax.experimental.pallas import tpu as pltpu
from jax.experimental.pallas import tpu_sc as plsc

N = 20000
G = 128
B = 8
RPW = 640
LAST_BASE = N - RPW
UF = 4
SEG = G + 2 * UF
NSEG = B * SEG // 16 + 1
CHUNK = 128
NCH = RPW // CHUNK


def _log_f32(x):
    bits = lax.bitcast_convert_type(x, jnp.int32)
    e = (bits >> 23) - 127
    mb = (bits & 0x007FFFFF) | 0x3F800000
    m = lax.bitcast_convert_type(mb, jnp.float32)
    big = m > 1.41421356
    m = jnp.where(big, m * 0.5, m)
    ef = e.astype(jnp.float32) + jnp.where(big, 1.0, 0.0)
    s = (m - 1.0) / (m + 1.0)
    t = s * s
    p = 2.0 + t * (0.6666666666 + t * (0.4 + t * 0.2857142857))
    return ef * 0.6931471805599453 + s * p


def _body(rois_hbm, rb_hbm, gt_hbm, gb_hbm,
          lab_hbm, dl_hbm, bw_hbm,
          roiq0_v, roiq1_v, rb_v, gts_v, gb_v,
          gx1_v, gy1_v, gx2_v, gy2_v, glb_v,
          plist_v, gcnt_v,
          x1s_v, y1s_v, x2s_v, y2s_v, ags_v, lbs_v,
          lab_v, dlq0_v, dlq1_v, bwq0_v, bwq1_v, fill_s,
          rsem0, rsem1, dsem0, dsem1, wsem0, wsem1):
    wid = lax.axis_index("s") * 2 + lax.axis_index("c")
    base = jnp.minimum(wid * RPW, LAST_BASE)
    pltpu.sync_copy(rb_hbm.at[pl.ds(base, RPW)], rb_v)
    pltpu.sync_copy(gt_hbm, gts_v)
    pltpu.sync_copy(gb_hbm, gb_v)

    iota = lax.iota(jnp.int32, 16)
    zero = jnp.zeros((16,), jnp.float32)
    izero = jnp.zeros((16,), jnp.int32)
    ione = jnp.ones((16,), jnp.int32)
    c0 = izero
    c1 = ione
    c2 = jnp.full((16,), 2, jnp.int32)
    c3 = jnp.full((16,), 3, jnp.int32)
    c4 = jnp.full((16,), 4, jnp.int32)

    for c in range(G // 16):
        sl = pl.ds(c * 16, 16)
        rows = iota + c * 16
        x1 = plsc.load_gather(gts_v, [rows, c0])
        y1 = plsc.load_gather(gts_v, [rows, c1])
        x2 = plsc.load_gather(gts_v, [rows, c2])
        y2 = plsc.load_gather(gts_v, [rows, c3])
        gx1_v[sl] = x1
        gy1_v[sl] = y1
        gx2_v[sl] = x2
        gy2_v[sl] = y2
        glb_v[sl] = plsc.load_gather(gts_v, [rows, c4])
    zsl = pl.ds(G, 16)
    gx1_v[zsl] = zero
    gy1_v[zsl] = zero
    gx2_v[zsl] = zero
    gy2_v[zsl] = zero
    glb_v[zsl] = zero

    counts = izero
    for c in range(G // 16):
        gbc = gb_v[pl.ds(c * 16, 16)]
        for b in range(B):
            m = gbc == b
            cnt = plsc.all_reduce_population_count(m)
            counts = counts + jnp.where(iota == b, cnt, izero)
    gcnt_v[...] = counts
    mmax = jnp.max(counts)
    stride = ((mmax + (UF - 1)) // UF) * UF

    def initp(i, carry):
        plist_v[pl.ds(i * 16, 16)] = jnp.full((16,), G, jnp.int32)
        return carry
    lax.fori_loop(0, NSEG, initp, 0)
    for b in range(B):
        fill_s[b] = b * stride
    for c in range(G // 16):
        gbc = gb_v[pl.ds(c * 16, 16)]
        lanes = iota + c * 16
        for b in range(B):
            m = gbc == b
            ones = jnp.where(m, ione, izero)
            pc = plsc.cumsum(ones)
            fb = fill_s[b]
            plsc.store_scatter(plist_v, [pc - 1 + fb], lanes, mask=m)
            fill_s[b] = fb + jnp.max(pc)

    def permute(i, carry):
        sl = pl.ds(i * 16, 16)
        gi = plist_v[sl]
        x1 = plsc.load_gather(gx1_v, [gi])
        y1 = plsc.load_gather(gy1_v, [gi])
        x2 = plsc.load_gather(gx2_v, [gi])
        y2 = plsc.load_gather(gy2_v, [gi])
        x1s_v[sl] = x1
        y1s_v[sl] = y1
        x2s_v[sl] = x2
        y2s_v[sl] = y2
        lbs_v[sl] = plsc.load_gather(glb_v, [gi])
        ags_v[sl] = (x2 - x1) * (y2 - y1)
        return carry
    lax.fori_loop(0, NSEG, permute, 0)

    stride_v = jnp.broadcast_to(stride, (16,))

    rbufs = (roiq0_v, roiq1_v)
    dbufs = (dlq0_v, dlq1_v)
    wbufs = (bwq0_v, bwq1_v)
    rsems = (rsem0, rsem1)
    dsems = (dsem0, dsem1)
    wsems = (wsem0, wsem1)
    rdesc = [None, None]
    ddesc = [None, None]
    wdesc = [None, None]

    rdesc[0] = pltpu.async_copy(
        rois_hbm.at[pl.ds(base, CHUNK)], rbufs[0], rsems[0])
    for q in range(NCH):
        A = q & 1
        if q + 1 < NCH:
            rdesc[1 - A] = pltpu.async_copy(
                rois_hbm.at[pl.ds(base + (q + 1) * CHUNK, CHUNK)],
                rbufs[1 - A], rsems[1 - A])
        rdesc[A].wait()
        if q >= 2:
            ddesc[A].wait()
            wdesc[A].wait()
        roiq_v = rbufs[A]
        dlq_v = dbufs[A]
        bwq_v = wbufs[A]

        def group(s, carry, q=q, roiq_v=roiq_v, dlq_v=dlq_v, bwq_v=bwq_v):
            rbase = q * CHUNK + s * 16
            rloc = iota + s * 16
            rx1 = plsc.load_gather(roiq_v, [rloc, c0])
            ry1 = plsc.load_gather(roiq_v, [rloc, c1])
            rx2 = plsc.load_gather(roiq_v, [rloc, c2])
            ry2 = plsc.load_gather(roiq_v, [rloc, c3])
            rb = rb_v[pl.ds(rbase, 16)]
            ar = (rx2 - rx1) * (ry2 - ry1)
            seg = rb * stride_v
            gseg_cnt = plsc.load_gather(gcnt_v, [rb])
            tmax = jnp.max(gseg_cnt)

            def gstep(i, st):
                ib, db, ab = st
                t0 = i * UF
                for k in range(UF):
                    pos = seg + jnp.broadcast_to(t0 + k, (16,))
                    gx1 = plsc.load_gather(x1s_v, [pos])
                    gy1 = plsc.load_gather(y1s_v, [pos])
                    gx2 = plsc.load_gather(x2s_v, [pos])
                    gy2 = plsc.load_gather(y2s_v, [pos])
                    ag = plsc.load_gather(ags_v, [pos])
                    iw = jnp.maximum(
                        jnp.minimum(rx2, gx2) - jnp.maximum(rx1, gx1), 0.0)
                    ih = jnp.maximum(
                        jnp.minimum(ry2, gy2) - jnp.maximum(ry1, gy1), 0.0)
                    inter = iw * ih
                    den = (ar + ag) - inter
                    upd = inter * db > ib * den
                    ib = jnp.where(upd, inter, ib)
                    db = jnp.where(upd, den, db)
                    ab = jnp.where(upd, pos, ab)
                return (ib, db, ab)

            nsteps = (tmax + (UF - 1)) // UF
            ib, db, ab = lax.fori_loop(
                0, nsteps, gstep,
                (zero, jnp.ones((16,), jnp.float32), izero))

            fg = (ib + ib) >= db
            mx1 = plsc.load_gather(x1s_v, [ab])
            my1 = plsc.load_gather(y1s_v, [ab])
            mx2 = plsc.load_gather(x2s_v, [ab])
            my2 = plsc.load_gather(y2s_v, [ab])
            mlb = plsc.load_gather(lbs_v, [ab])
            pw = rx2 - rx1
            ph = ry2 - ry1
            pcx = rx1 + 0.5 * pw
            pcy = ry1 + 0.5 * ph
            gw = mx2 - mx1
            gh = my2 - my1
            gcx = mx1 + 0.5 * gw
            gcy = my1 + 0.5 * gh
            pwe = pw + 1e-12
            phe = ph + 1e-12
            dx = (gcx - pcx) / pwe
            dy = (gcy - pcy) / phe
            dw = _log_f32(gw / pwe + 1e-12)
            dh = _log_f32(gh / phe + 1e-12)
            lab_v[pl.ds(rbase, 16)] = jnp.where(fg, mlb, zero)
            plsc.store_scatter(dlq_v, [rloc, c0], jnp.where(fg, dx, zero))
            plsc.store_scatter(dlq_v, [rloc, c1], jnp.where(fg, dy, zero))
            plsc.store_scatter(dlq_v, [rloc, c2], jnp.where(fg, dw, zero))
            plsc.store_scatter(dlq_v, [rloc, c3], jnp.where(fg, dh, zero))
            one = jnp.where(fg, jnp.ones((16,), jnp.float32), zero)
            plsc.store_scatter(bwq_v, [rloc, c0], one)
            plsc.store_scatter(bwq_v, [rloc, c1], one)
            plsc.store_scatter(bwq_v, [rloc, c2], one)
            plsc.store_scatter(bwq_v, [rloc, c3], one)
            return carry

        lax.fori_loop(0, CHUNK // 16, group, 0)
        crow = base + q * CHUNK
        ddesc[A] = pltpu.async_copy(dlq_v, dl_hbm.at[pl.ds(crow, CHUNK)], dsems[A])
        wdesc[A] = pltpu.async_copy(bwq_v, bw_hbm.at[pl.ds(crow, CHUNK)], wsems[A])

    ddesc[(NCH - 1) & 1].wait()
    wdesc[(NCH - 1) & 1].wait()
    ddesc[(NCH - 2) & 1].wait()
    wdesc[(NCH - 2) & 1].wait()

    pltpu.sync_copy(lab_v, lab_hbm.at[pl.ds(base, RPW)])


def kernel(rois, roi_batch_inds, gt_boxes, gt_batch_inds):
    mesh = plsc.VectorSubcoreMesh(core_axis_name="c", subcore_axis_name="s")
    run = pl.kernel(
        _body,
        out_type=(jax.ShapeDtypeStruct((N,), jnp.float32),
                  jax.ShapeDtypeStruct((N, 4), jnp.float32),
                  jax.ShapeDtypeStruct((N, 4), jnp.float32)),
        mesh=mesh,
        compiler_params=pltpu.CompilerParams(needs_layout_passes=False),
        scratch_types=[
            pltpu.VMEM((CHUNK, 5), jnp.float32),
            pltpu.VMEM((CHUNK, 5), jnp.float32),
            pltpu.VMEM((RPW,), jnp.int32),
            pltpu.VMEM((G, 5), jnp.float32),
            pltpu.VMEM((G,), jnp.int32),
            pltpu.VMEM((G + 16,), jnp.float32),
            pltpu.VMEM((G + 16,), jnp.float32),
            pltpu.VMEM((G + 16,), jnp.float32),
            pltpu.VMEM((G + 16,), jnp.float32),
            pltpu.VMEM((G + 16,), jnp.float32),
            pltpu.VMEM((NSEG * 16,), jnp.int32),
            pltpu.VMEM((16,), jnp.int32),
            pltpu.VMEM((NSEG * 16,), jnp.float32),
            pltpu.VMEM((NSEG * 16,), jnp.float32),
            pltpu.VMEM((NSEG * 16,), jnp.float32),
            pltpu.VMEM((NSEG * 16,), jnp.float32),
            pltpu.VMEM((NSEG * 16,), jnp.float32),
            pltpu.VMEM((NSEG * 16,), jnp.float32),
            pltpu.VMEM((RPW,), jnp.float32),
            pltpu.VMEM((CHUNK, 4), jnp.float32),
            pltpu.VMEM((CHUNK, 4), jnp.float32),
            pltpu.VMEM((CHUNK, 4), jnp.float32),
            pltpu.VMEM((CHUNK, 4), jnp.float32),
            pltpu.SMEM((B,), jnp.int32),
            pltpu.SemaphoreType.DMA,
            pltpu.SemaphoreType.DMA,
            pltpu.SemaphoreType.DMA,
            pltpu.SemaphoreType.DMA,
            pltpu.SemaphoreType.DMA,
            pltpu.SemaphoreType.DMA,
        ],
    )
    lab, dl, bw = run(rois, roi_batch_inds, gt_boxes, gt_batch_inds)
    return lab, dl, bw
